# baseline (speedup 1.0000x reference)
import jax
import jax.numpy as jnp
from jax import lax
from jax.experimental import pallas as pl

VOCAB = 1000
HIDDEN = 16
BM = 512


def _body(ids_ref, tab_ref, w_ref, b_ref, out_ref):
    ids = ids_ref[0]
    onehot = (ids == lax.broadcasted_iota(jnp.int32, (BM, VOCAB), 1)).astype(
        jnp.float32)
    x = jnp.dot(onehot, tab_ref[...], preferred_element_type=jnp.float32)
    y = lax.dot_general(x, w_ref[...], (((1,), (1,)), ((), ())),
                        preferred_element_type=jnp.float32)
    out_ref[...] = y + b_ref[...]


def kernel(input_ids, embed_table, W, b):
    B, S = input_ids.shape
    n = B * S
    grid = n // BM
    ids3 = input_ids.reshape(grid, BM, 1).astype(jnp.int32)
    b2 = b.reshape(1, VOCAB)
    out = pl.pallas_call(
        _body,
        grid=(grid,),
        in_specs=[
            pl.BlockSpec((1, BM, 1), lambda i: (i, 0, 0)),
            pl.BlockSpec((VOCAB, HIDDEN), lambda i: (0, 0)),
            pl.BlockSpec((VOCAB, HIDDEN), lambda i: (0, 0)),
            pl.BlockSpec((1, VOCAB), lambda i: (0, 0)),
        ],
        out_specs=pl.BlockSpec((BM, VOCAB), lambda i: (i, 0)),
        out_shape=jax.ShapeDtypeStruct((n, VOCAB), jnp.float32),
    )(ids3, embed_table, W, b2)
    return out.reshape(B, S, VOCAB)
